# Initial kernel scaffold; baseline (speedup 1.0000x reference)
import jax
import jax.numpy as jnp
from jax.experimental import pallas as pl


def kernel(h, edge_index, edge_types, W_attn, W_lin):
    raise NotImplementedError("write your pallas kernel here")


# speedup vs baseline: 4.6447x; 4.6447x over previous
import functools

import jax
import jax.numpy as jnp
from jax import lax
from jax.experimental import pallas as pl
from jax.experimental.pallas import tpu as pltpu
from jax.experimental.pallas import tpu_sc as plsc

N_NODES = 10000
N_EDGES = 320000
D_FEAT = 128
OUT_DIM = 128

NC = 2
NS = 16
NW = NC * NS
CHUNK = 128
CHUNKS_PER_W = 80
EDGES_PER_W = CHUNKS_PER_W * CHUNK
E_PAD = NW * EDGES_PER_W
ACC_ROWS = 10240
ROWS_PER_TILE = ACC_ROWS // NS
DUMMY_ROW = N_NODES


def _sc_body(h_hbm, zeros_hbm, sidx_hbm, didx_hbm, out_hbm,
             sidx, didx, rows, acc, sem):
    cid = lax.axis_index("c")
    sid = lax.axis_index("s")
    wid = sid * NC + cid

    base = sid * ROWS_PER_TILE
    pltpu.sync_copy(zeros_hbm, acc.at[pl.ds(base, ROWS_PER_TILE)])

    pltpu.sync_copy(sidx_hbm.at[wid], sidx)
    pltpu.sync_copy(didx_hbm.at[wid], didx)
    plsc.subcore_barrier()

    def chunk_step(j, carry):
        pltpu.async_copy(h_hbm.at[sidx.at[j]], rows, sem).wait()
        pltpu.sync_copy(rows, acc.at[didx.at[j]], add=True)
        return carry

    lax.fori_loop(0, CHUNKS_PER_W, chunk_step, 0)
    plsc.subcore_barrier()

    out_base = cid * ACC_ROWS + base
    pltpu.sync_copy(acc.at[pl.ds(base, ROWS_PER_TILE)],
                    out_hbm.at[pl.ds(out_base, ROWS_PER_TILE)])


@functools.cache
def _sc_aggregate():
    return pl.kernel(
        _sc_body,
        out_type=jax.ShapeDtypeStruct((NC * ACC_ROWS, D_FEAT), jnp.float32),
        mesh=plsc.VectorSubcoreMesh(core_axis_name="c", subcore_axis_name="s"),
        scratch_types=[
            pltpu.VMEM((CHUNKS_PER_W, CHUNK), jnp.int32),
            pltpu.VMEM((CHUNKS_PER_W, CHUNK), jnp.int32),
            pltpu.VMEM((CHUNK, D_FEAT), jnp.float32),
            pltpu.VMEM_SHARED((ACC_ROWS, D_FEAT), jnp.float32),
            pltpu.SemaphoreType.DMA,
        ],
    )


def _mm_body(p_ref, w_ref, o_ref):
    a = p_ref[0] + p_ref[1]
    o_ref[...] = jnp.dot(a, w_ref[...],
                         preferred_element_type=jnp.float32,
                         precision=jax.lax.Precision.HIGHEST)


def _tc_matmul(partial, w_t):
    blk = 1000
    return pl.pallas_call(
        _mm_body,
        grid=(N_NODES // blk,),
        in_specs=[
            pl.BlockSpec((2, blk, D_FEAT), lambda i: (0, i, 0)),
            pl.BlockSpec((D_FEAT, OUT_DIM), lambda i: (0, 0)),
        ],
        out_specs=pl.BlockSpec((blk, OUT_DIM), lambda i: (i, 0)),
        out_shape=jax.ShapeDtypeStruct((N_NODES, OUT_DIM), jnp.float32),
    )(partial, w_t)


def kernel(h, edge_index, edge_types, W_attn, W_lin):
    src = edge_index[0].astype(jnp.int32)
    dst = edge_index[1].astype(jnp.int32)
    pad = E_PAD - N_EDGES
    src3 = jnp.concatenate(
        [src, jnp.zeros((pad,), jnp.int32)]).reshape(NW, CHUNKS_PER_W, CHUNK)
    dst3 = jnp.concatenate(
        [dst, jnp.full((pad,), DUMMY_ROW, jnp.int32)]).reshape(
            NW, CHUNKS_PER_W, CHUNK)
    zeros = jnp.zeros((ROWS_PER_TILE, D_FEAT), jnp.float32)

    partial = _sc_aggregate()(h, zeros, src3, dst3)
    return _tc_matmul(partial.reshape(NC, ACC_ROWS, D_FEAT), W_lin.T)


# speedup vs baseline: 5.3893x; 1.1603x over previous
import functools

import jax
import jax.numpy as jnp
from jax import lax
from jax.experimental import pallas as pl
from jax.experimental.pallas import tpu as pltpu
from jax.experimental.pallas import tpu_sc as plsc

N_NODES = 10000
N_EDGES = 320000
D_FEAT = 128
OUT_DIM = 128

NC = 2
NS = 16
NW = NC * NS
CHUNK = 128
CHUNKS_PER_W = 80
GROUP = 8
NG = CHUNKS_PER_W // GROUP
EDGES_PER_W = CHUNKS_PER_W * CHUNK
E_PAD = NW * EDGES_PER_W
ACC_ROWS = 10240
ROWS_PER_TILE = ACC_ROWS // NS
DUMMY_ROW = N_NODES


def _sc_body(h_hbm, zeros_hbm, sidx_hbm, didx_hbm, out_hbm,
             sidx, didx, rows, acc, sems, semi):
    cid = lax.axis_index("c")
    sid = lax.axis_index("s")
    wid = sid * NC + cid

    base = sid * ROWS_PER_TILE
    pltpu.sync_copy(zeros_hbm, acc.at[pl.ds(base, ROWS_PER_TILE)])

    pltpu.sync_copy(didx_hbm.at[wid], didx)
    pltpu.sync_copy(sidx_hbm.at[wid, pl.ds(0, GROUP)], sidx.at[0])
    pltpu.async_copy(sidx_hbm.at[wid, pl.ds(GROUP, GROUP)], sidx.at[1], semi)
    plsc.subcore_barrier()

    for b in (0, 1):
        pltpu.async_copy(h_hbm.at[sidx.at[0, b]], rows.at[b], sems.at[b])

    def chunk_step(j, carry):
        pb = j % 2
        g = j // GROUP
        pltpu.make_async_copy(
            h_hbm.at[sidx.at[0, 0]], rows.at[pb], sems.at[pb]).wait()
        pltpu.sync_copy(rows.at[pb], acc.at[didx.at[j]], add=True)

        @pl.when((j % GROUP == GROUP - 2) & (j + 2 < CHUNKS_PER_W))
        def _():
            pltpu.make_async_copy(
                sidx_hbm.at[wid, pl.ds(0, GROUP)],
                sidx.at[(g + 1) % 2], semi).wait()

        @pl.when(j + 2 < CHUNKS_PER_W)
        def _():
            j2 = j + 2
            pltpu.async_copy(
                h_hbm.at[sidx.at[(j2 // GROUP) % 2, j2 % GROUP]],
                rows.at[pb], sems.at[pb])

        @pl.when((j % GROUP == GROUP - 1) & (g + 2 < NG))
        def _():
            pltpu.async_copy(
                sidx_hbm.at[wid, pl.ds((g + 2) * GROUP, GROUP)],
                sidx.at[g % 2], semi)
        return carry

    lax.fori_loop(0, CHUNKS_PER_W, chunk_step, 0)
    plsc.subcore_barrier()

    out_base = cid * ACC_ROWS + base
    pltpu.sync_copy(acc.at[pl.ds(base, ROWS_PER_TILE)],
                    out_hbm.at[pl.ds(out_base, ROWS_PER_TILE)])


@functools.cache
def _sc_aggregate():
    return pl.kernel(
        _sc_body,
        out_type=jax.ShapeDtypeStruct((NC * ACC_ROWS, D_FEAT), jnp.float32),
        mesh=plsc.VectorSubcoreMesh(core_axis_name="c", subcore_axis_name="s"),
        scratch_types=[
            pltpu.VMEM((2, GROUP, CHUNK), jnp.int32),
            pltpu.VMEM((CHUNKS_PER_W, CHUNK), jnp.int32),
            pltpu.VMEM((2, CHUNK, D_FEAT), jnp.float32),
            pltpu.VMEM_SHARED((ACC_ROWS, D_FEAT), jnp.float32),
            pltpu.SemaphoreType.DMA((2,)),
            pltpu.SemaphoreType.DMA,
        ],
    )


def _mm_body(p_ref, w_ref, o_ref):
    a = p_ref[0] + p_ref[1]
    o_ref[...] = jnp.dot(a, w_ref[...],
                         preferred_element_type=jnp.float32,
                         precision=jax.lax.Precision.HIGHEST)


def _tc_matmul(partial, w_t):
    blk = 1000
    return pl.pallas_call(
        _mm_body,
        grid=(N_NODES // blk,),
        in_specs=[
            pl.BlockSpec((2, blk, D_FEAT), lambda i: (0, i, 0)),
            pl.BlockSpec((D_FEAT, OUT_DIM), lambda i: (0, 0)),
        ],
        out_specs=pl.BlockSpec((blk, OUT_DIM), lambda i: (i, 0)),
        out_shape=jax.ShapeDtypeStruct((N_NODES, OUT_DIM), jnp.float32),
    )(partial, w_t)


def kernel(h, edge_index, edge_types, W_attn, W_lin):
    src = edge_index[0].astype(jnp.int32)
    dst = edge_index[1].astype(jnp.int32)
    pad = E_PAD - N_EDGES
    src3 = jnp.concatenate(
        [src, jnp.zeros((pad,), jnp.int32)]).reshape(NW, CHUNKS_PER_W, CHUNK)
    dst3 = jnp.concatenate(
        [dst, jnp.full((pad,), DUMMY_ROW, jnp.int32)]).reshape(
            NW, CHUNKS_PER_W, CHUNK)
    zeros = jnp.zeros((ROWS_PER_TILE, D_FEAT), jnp.float32)

    partial = _sc_aggregate()(h, zeros, src3, dst3)
    return _tc_matmul(partial.reshape(NC, ACC_ROWS, D_FEAT), W_lin.T)
